# baseline (speedup 1.0000x reference)
import jax
import jax.numpy as jnp
from jax.experimental import pallas as pl
from jax.experimental.pallas import tpu as pltpu

_CB = 128


def _css_body(p_ref, f_ref, o_ref, wt_ref):
    cb = pl.program_id(1)

    @pl.when(cb == 0)
    def _():
        p = p_ref[0]
        m = jnp.max(p, axis=1, keepdims=True)
        e = jnp.exp(p - m)
        z = jnp.sum(e, axis=1, keepdims=True)
        wt_ref[...] = jnp.transpose(e * (1.0 / z), (1, 0))

    o_ref[0] = jax.lax.dot_general(
        f_ref[0], wt_ref[...], (((1,), (0,)), ((), ())),
        preferred_element_type=jnp.float32)


def kernel(feats, probs):
    B, K, H, W = probs.shape
    C = feats.shape[1]
    HW = H * W
    f = feats.reshape(B, C, HW)
    p = probs.reshape(B, K, HW)
    out = pl.pallas_call(
        _css_body,
        grid=(B, C // _CB),
        in_specs=[
            pl.BlockSpec((1, K, HW), lambda b, cb: (b, 0, 0)),
            pl.BlockSpec((1, _CB, HW), lambda b, cb: (b, cb, 0)),
        ],
        out_specs=pl.BlockSpec((1, _CB, K), lambda b, cb: (b, cb, 0)),
        out_shape=jax.ShapeDtypeStruct((B, C, K), jnp.float32),
        scratch_shapes=[
            pltpu.VMEM((HW, K), jnp.float32),
        ],
        compiler_params=pltpu.CompilerParams(
            dimension_semantics=("parallel", "arbitrary"),
            vmem_limit_bytes=48 * 1024 * 1024,
        ),
        name="css_softmax_pool",
    )(p, f)
    return out[..., None]
